# baseline (speedup 1.0000x reference)
import functools

import jax
import jax.numpy as jnp
from jax import lax
from jax.experimental import pallas as pl
from jax.experimental.pallas import tpu as pltpu
from jax.experimental.pallas import tpu_sc as plsc

_NC = 2
_NS = 16
_NW = _NC * _NS
_LANES = 16

_EPS = 1e-5


def _rsqrt16(x):
    i = lax.bitcast_convert_type(x, jnp.int32)
    i = jnp.int32(0x5F3759DF) - lax.shift_right_logical(i, 1)
    y = lax.bitcast_convert_type(i, jnp.float32)
    half_x = 0.5 * x
    for _ in range(3):
        y = y * (1.5 - half_x * y * y)
    return y


def _lane_sum(x, perms):
    for perm in perms:
        x = x + x.at[perm].get(mode="promise_in_bounds")
    return x


def _make_norm_kernel(L, T, C, rows_per_chunk):
    n_chunks = L // rows_per_chunk
    chunks_per_w = -(-n_chunks // _NW)
    sub = C // _LANES
    crows = rows_per_chunk * T

    mesh = plsc.VectorSubcoreMesh(core_axis_name="c", subcore_axis_name="s")

    @functools.partial(
        pl.kernel,
        mesh=mesh,
        out_type=jax.ShapeDtypeStruct((L * T, C), jnp.float32),
        scratch_types=[
            pltpu.VMEM((crows, C), jnp.float32),
            pltpu.VMEM((C,), jnp.float32),
            pltpu.VMEM((C,), jnp.float32),
        ],
    )
    def norm_kernel(tab_hbm, gamma_hbm, beta_hbm, out_hbm, buf_v, gam_v, bet_v):
        wid = lax.axis_index("s") * _NC + lax.axis_index("c")
        pltpu.sync_copy(gamma_hbm, gam_v)
        pltpu.sync_copy(beta_hbm, bet_v)
        lane = lax.iota(jnp.int32, _LANES)
        perms = tuple(lane ^ step for step in (8, 4, 2, 1))
        inv_n = jnp.float32(1.0 / C)

        def normalize_crow(row):
            def acc(j, carry):
                s, q = carry
                v = buf_v[row, pl.ds(j * _LANES, _LANES)]
                return s + v, q + v * v

            zeros = jnp.zeros((_LANES,), jnp.float32)
            s, q = lax.fori_loop(0, sub, acc, (zeros, zeros))
            mean_v = _lane_sum(s, perms) * inv_n
            var_v = _lane_sum(q, perms) * inv_n - mean_v * mean_v
            rstd_v = _rsqrt16(var_v + _EPS)

            def norm(j, carry):
                off = j * _LANES
                v = buf_v[row, pl.ds(off, _LANES)]
                g = gam_v[pl.ds(off, _LANES)]
                b = bet_v[pl.ds(off, _LANES)]
                buf_v[row, pl.ds(off, _LANES)] = (v - mean_v) * rstd_v * g + b
                return carry

            lax.fori_loop(0, sub, norm, 0)

        for i in range(chunks_per_w):
            chunk = wid + _NW * i

            @pl.when(chunk < n_chunks)
            def _():
                base = chunk * crows
                pltpu.sync_copy(tab_hbm.at[pl.ds(base, crows)], buf_v)

                def per_crow(row, carry):
                    normalize_crow(row)
                    return carry

                lax.fori_loop(0, crows, per_crow, 0)
                pltpu.sync_copy(buf_v, out_hbm.at[pl.ds(base, crows)])

    return norm_kernel


def _make_gather_kernel(L, T, C, B, crows_per_chunk):
    total = B * T
    per_w = total // _NW
    n_chunks = per_w // crows_per_chunk

    mesh = plsc.VectorSubcoreMesh(core_axis_name="c", subcore_axis_name="s")

    @functools.partial(
        pl.kernel,
        mesh=mesh,
        out_type=jax.ShapeDtypeStruct((total, C), jnp.float32),
        scratch_types=[
            pltpu.VMEM((n_chunks, crows_per_chunk), jnp.int32),
            pltpu.VMEM((crows_per_chunk, C), jnp.float32),
            pltpu.VMEM((crows_per_chunk, C), jnp.float32),
            pltpu.SemaphoreType.DMA,
            pltpu.SemaphoreType.DMA,
            pltpu.SemaphoreType.DMA,
            pltpu.SemaphoreType.DMA,
        ],
    )
    def gather_kernel(tab_hbm, idx_hbm, out_hbm, idx_v, buf0, buf1, si0, si1, so0, so1):
        wid = lax.axis_index("s") * _NC + lax.axis_index("c")
        base = wid * per_w
        pltpu.sync_copy(idx_hbm.at[wid], idx_v)

        bufs = (buf0, buf1)
        sin = (si0, si1)
        sout = (so0, so1)
        in_h = [None] * n_chunks
        out_h = [None] * n_chunks
        in_h[0] = pltpu.async_copy(tab_hbm.at[idx_v.at[0]], bufs[0], sin[0])
        for c in range(n_chunks):
            p = c % 2
            in_h[c].wait()
            if c + 1 < n_chunks:
                if c >= 1:
                    out_h[c - 1].wait()
                in_h[c + 1] = pltpu.async_copy(
                    tab_hbm.at[idx_v.at[c + 1]], bufs[1 - p], sin[1 - p]
                )
            out_h[c] = pltpu.async_copy(
                bufs[p],
                out_hbm.at[pl.ds(base + c * crows_per_chunk, crows_per_chunk)],
                sout[p],
            )
        out_h[n_chunks - 2].wait()
        out_h[n_chunks - 1].wait()

    return gather_kernel


def kernel(instructions, gamma, beta, idx_subject, idx_label):
    S, L, T, C = instructions.shape
    B = idx_label.shape[0]

    tab = jnp.reshape(instructions[idx_subject], (L * T, C))

    rows_a = 8
    norm_fn = _make_norm_kernel(L, T, C, rows_a)
    norm_tab = norm_fn(tab, gamma, beta)

    crows_b = 80
    idxc = idx_label.astype(jnp.int32)
    gidx = (idxc[:, None] * T + jnp.arange(T, dtype=jnp.int32)[None, :]).reshape(
        _NW, (B * T) // (_NW * crows_b), crows_b
    )
    gather_fn = _make_gather_kernel(L, T, C, B, crows_b)
    out = gather_fn(norm_tab, gidx)

    return jnp.reshape(out, (B, T, C))


# speedup vs baseline: 1.0894x; 1.0894x over previous
import functools

import jax
import jax.numpy as jnp
from jax import lax
from jax.experimental import pallas as pl
from jax.experimental.pallas import tpu as pltpu
from jax.experimental.pallas import tpu_sc as plsc

_NC = 2
_NS = 16
_NW = _NC * _NS
_LANES = 16

_EPS = 1e-5
_ROWS = 4


def _rsqrt16(x):
    i = lax.bitcast_convert_type(x, jnp.int32)
    i = jnp.int32(0x5F3759DF) - lax.shift_right_logical(i, 1)
    y = lax.bitcast_convert_type(i, jnp.float32)
    half_x = 0.5 * x
    for _ in range(3):
        y = y * (1.5 - half_x * y * y)
    return y


def _lane_sum(x, perms):
    for perm in perms:
        x = x + x.at[perm].get(mode="promise_in_bounds")
    return x


def _tree_add(vs):
    while len(vs) > 1:
        vs = [a + b for a, b in zip(vs[::2], vs[1::2])]
    return vs[0]


def _make_kernel(L, T, C, B):
    D = T * C
    per_w = B // _NW
    n_chunks = per_w // _ROWS
    sub = C // _LANES

    mesh = plsc.VectorSubcoreMesh(core_axis_name="c", subcore_axis_name="s")

    @functools.partial(
        pl.kernel,
        mesh=mesh,
        out_type=jax.ShapeDtypeStruct((B, T, C), jnp.float32),
        scratch_types=[
            pltpu.VMEM((n_chunks, _ROWS), jnp.int32),
            pltpu.VMEM((_ROWS, D), jnp.float32),
            pltpu.VMEM((_ROWS, D), jnp.float32),
            pltpu.VMEM((_ROWS, T, C), jnp.float32),
            pltpu.VMEM((_ROWS, T, C), jnp.float32),
            pltpu.VMEM((C,), jnp.float32),
            pltpu.VMEM((C,), jnp.float32),
            pltpu.SemaphoreType.DMA,
            pltpu.SemaphoreType.DMA,
            pltpu.SemaphoreType.DMA,
            pltpu.SemaphoreType.DMA,
        ],
    )
    def fused_kernel(tab_hbm, idx_hbm, gamma_hbm, beta_hbm, out_hbm,
                     idx_v, in0, in1, st0, st1, gam_v, bet_v,
                     si0, si1, so0, so1):
        wid = lax.axis_index("s") * _NC + lax.axis_index("c")
        base = wid * per_w
        pltpu.sync_copy(idx_hbm.at[wid], idx_v)
        pltpu.sync_copy(gamma_hbm, gam_v)
        pltpu.sync_copy(beta_hbm, bet_v)
        lane = lax.iota(jnp.int32, _LANES)
        perms = tuple(lane ^ step for step in (8, 4, 2, 1))
        inv_n = jnp.float32(1.0 / C)

        inb = (in0, in1)
        stb = (st0, st1)
        sin = (si0, si1)
        sout = (so0, so1)

        def compute(src, dst):
            def per_row(r, carry):
                def per_group(t, carry2):
                    off = t * C
                    vs = [src[r, pl.ds(off + j * _LANES, _LANES)] for j in range(sub)]
                    s = _tree_add(vs)
                    q = _tree_add([v * v for v in vs])
                    mean_v = _lane_sum(s, perms) * inv_n
                    var_v = _lane_sum(q, perms) * inv_n - mean_v * mean_v
                    rstd_v = _rsqrt16(var_v + _EPS)
                    for j in range(sub):
                        g = gam_v[pl.ds(j * _LANES, _LANES)]
                        b = bet_v[pl.ds(j * _LANES, _LANES)]
                        dst[r, t, pl.ds(j * _LANES, _LANES)] = (
                            (vs[j] - mean_v) * rstd_v * g + b
                        )
                    return carry2

                return lax.fori_loop(0, T, per_group, carry)

            lax.fori_loop(0, _ROWS, per_row, 0)

        def wait_in(k):
            pltpu.make_async_copy(tab_hbm.at[idx_v.at[0]], inb[k], sin[k]).wait()

        def wait_out(k):
            pltpu.make_async_copy(
                stb[k], out_hbm.at[pl.ds(0, _ROWS)], sout[k]
            ).wait()

        def chunk_pair(cbase, first, last):
            for k in (0, 1):
                c = cbase + k
                wait_in(k)
                if not first:
                    wait_out(k)
                compute(inb[k], stb[k])
                pltpu.async_copy(
                    stb[k], out_hbm.at[pl.ds(base + c * _ROWS, _ROWS)], sout[k]
                )
                if not last:
                    pltpu.async_copy(
                        tab_hbm.at[idx_v.at[c + 2]], inb[k], sin[k]
                    )

        pltpu.async_copy(tab_hbm.at[idx_v.at[0]], in0, si0)
        pltpu.async_copy(tab_hbm.at[idx_v.at[1]], in1, si1)

        chunk_pair(0, first=True, last=False)

        def body(i, carry):
            chunk_pair(2 * i, first=False, last=False)
            return carry

        lax.fori_loop(1, n_chunks // 2 - 1, body, 0)

        chunk_pair(n_chunks - 2, first=False, last=True)
        wait_out(0)
        wait_out(1)

    return fused_kernel


def kernel(instructions, gamma, beta, idx_subject, idx_label):
    S, L, T, C = instructions.shape
    B = idx_label.shape[0]

    tab = jnp.reshape(instructions[idx_subject], (L, T * C))
    per_w = B // _NW
    idx = jnp.reshape(idx_label.astype(jnp.int32), (_NW, per_w // _ROWS, _ROWS))
    fn = _make_kernel(L, T, C, B)
    return fn(tab, idx, gamma, beta)


# speedup vs baseline: 1.3516x; 1.2407x over previous
import functools

import jax
import jax.numpy as jnp
from jax import lax
from jax.experimental import pallas as pl
from jax.experimental.pallas import tpu as pltpu
from jax.experimental.pallas import tpu_sc as plsc

_NC = 2
_NS = 16
_NW = _NC * _NS
_LANES = 16

_EPS = 1e-5
_ROWS = 4


def _rsqrt16(x):
    i = lax.bitcast_convert_type(x, jnp.int32)
    i = jnp.int32(0x5F3759DF) - lax.shift_right_logical(i, 1)
    y = lax.bitcast_convert_type(i, jnp.float32)
    half_x = 0.5 * x
    for _ in range(3):
        y = y * (1.5 - half_x * y * y)
    return y


def _lane_sum(x, perms):
    for perm in perms:
        x = x + x.at[perm].get(mode="promise_in_bounds")
    return x


def _tree_add(vs):
    while len(vs) > 1:
        vs = [a + b for a, b in zip(vs[::2], vs[1::2])]
    return vs[0]


def _make_kernel(L, T, C, B):
    D = T * C
    CL = C // 128
    per_w = B // _NW
    n_chunks = per_w // _ROWS
    sub = C // _LANES

    mesh = plsc.VectorSubcoreMesh(core_axis_name="c", subcore_axis_name="s")

    @functools.partial(
        pl.kernel,
        mesh=mesh,
        out_type=jax.ShapeDtypeStruct((B, T, C), jnp.float32),
        scratch_types=[
            pltpu.VMEM((n_chunks, _ROWS), jnp.int32),
            pltpu.VMEM((_ROWS, D // 128, 128), jnp.float32),
            pltpu.VMEM((_ROWS, D // 128, 128), jnp.float32),
            pltpu.VMEM((_ROWS, T, C), jnp.float32),
            pltpu.VMEM((_ROWS, T, C), jnp.float32),
            pltpu.VMEM((C,), jnp.float32),
            pltpu.VMEM((C,), jnp.float32),
            pltpu.SemaphoreType.DMA,
            pltpu.SemaphoreType.DMA,
            pltpu.SemaphoreType.DMA,
            pltpu.SemaphoreType.DMA,
        ],
    )
    def fused_kernel(tab_hbm, idx_hbm, gamma_hbm, beta_hbm, out_hbm,
                     idx_v, in0, in1, st0, st1, gam_v, bet_v,
                     si0, si1, so0, so1):
        wid = lax.axis_index("s") * _NC + lax.axis_index("c")
        base = wid * per_w
        pltpu.sync_copy(idx_hbm.at[wid], idx_v)
        pltpu.sync_copy(gamma_hbm, gam_v)
        pltpu.sync_copy(beta_hbm, bet_v)
        lane = lax.iota(jnp.int32, _LANES)
        perms = tuple(lane ^ step for step in (8, 4, 2, 1))
        inv_n = jnp.float32(1.0 / C)

        inb = (in0, in1)
        stb = (st0, st1)
        sin = (si0, si1)
        sout = (so0, so1)

        def compute(src, dst):
            def per_row(r, carry):
                def per_pair(th, carry2):
                    u0 = (2 * th) * CL
                    u1 = u0 + CL
                    zeros = jnp.zeros((_LANES,), jnp.float32)
                    s0 = q0 = s1 = q1 = zeros

                    def acc(j, carry3):
                        a0, b0, a1, b1 = carry3
                        u = j // 8
                        col = (j % 8) * _LANES
                        v0 = src[r, u0 + u, pl.ds(col, _LANES)]
                        v1 = src[r, u1 + u, pl.ds(col, _LANES)]
                        return a0 + v0, b0 + v0 * v0, a1 + v1, b1 + v1 * v1

                    s0, q0, s1, q1 = lax.fori_loop(
                        0, sub, acc, (s0, q0, s1, q1), unroll=4
                    )
                    mean0 = _lane_sum(s0, perms) * inv_n
                    mean1 = _lane_sum(s1, perms) * inv_n
                    var0 = _lane_sum(q0, perms) * inv_n - mean0 * mean0
                    var1 = _lane_sum(q1, perms) * inv_n - mean1 * mean1
                    rstd0 = _rsqrt16(var0 + _EPS)
                    rstd1 = _rsqrt16(var1 + _EPS)

                    def norm(j, carry3):
                        off = j * _LANES
                        u = j // 8
                        col = (j % 8) * _LANES
                        g = gam_v[pl.ds(off, _LANES)]
                        b = bet_v[pl.ds(off, _LANES)]
                        v0 = src[r, u0 + u, pl.ds(col, _LANES)]
                        v1 = src[r, u1 + u, pl.ds(col, _LANES)]
                        dst[r, 2 * th, pl.ds(off, _LANES)] = (
                            (v0 - mean0) * rstd0 * g + b
                        )
                        dst[r, 2 * th + 1, pl.ds(off, _LANES)] = (
                            (v1 - mean1) * rstd1 * g + b
                        )
                        return carry3

                    lax.fori_loop(0, sub, norm, 0, unroll=4)
                    return carry2

                return lax.fori_loop(0, T // 2, per_pair, carry)

            lax.fori_loop(0, _ROWS, per_row, 0)

        def wait_in(k):
            pltpu.make_async_copy(tab_hbm.at[idx_v.at[0]], inb[k], sin[k]).wait()

        def wait_out(k):
            pltpu.make_async_copy(
                stb[k], out_hbm.at[pl.ds(0, _ROWS)], sout[k]
            ).wait()

        def chunk_pair(cbase, first, last):
            for k in (0, 1):
                c = cbase + k
                wait_in(k)
                if not first:
                    wait_out(k)
                compute(inb[k], stb[k])
                pltpu.async_copy(
                    stb[k], out_hbm.at[pl.ds(base + c * _ROWS, _ROWS)], sout[k]
                )
                if not last:
                    pltpu.async_copy(
                        tab_hbm.at[idx_v.at[c + 2]], inb[k], sin[k]
                    )

        pltpu.async_copy(tab_hbm.at[idx_v.at[0]], in0, si0)
        pltpu.async_copy(tab_hbm.at[idx_v.at[1]], in1, si1)

        chunk_pair(0, first=True, last=False)

        def body(i, carry):
            chunk_pair(2 * i, first=False, last=False)
            return carry

        lax.fori_loop(1, n_chunks // 2 - 1, body, 0)

        chunk_pair(n_chunks - 2, first=False, last=True)
        wait_out(0)
        wait_out(1)

    return fused_kernel


def kernel(instructions, gamma, beta, idx_subject, idx_label):
    S, L, T, C = instructions.shape
    B = idx_label.shape[0]

    tab = jnp.reshape(instructions[idx_subject], (L, (T * C) // 128, 128))
    per_w = B // _NW
    idx = jnp.reshape(idx_label.astype(jnp.int32), (_NW, per_w // _ROWS, _ROWS))
    fn = _make_kernel(L, T, C, B)
    return fn(tab, idx, gamma, beta)


# speedup vs baseline: 1.7672x; 1.3075x over previous
import functools

import jax
import jax.numpy as jnp
from jax import lax
from jax.experimental import pallas as pl
from jax.experimental.pallas import tpu as pltpu
from jax.experimental.pallas import tpu_sc as plsc

_NC = 2
_NS = 16
_NW = _NC * _NS
_LANES = 16

_EPS = 1e-5
_ROWS = 4


def _rsqrt16(x):
    i = lax.bitcast_convert_type(x, jnp.int32)
    i = jnp.int32(0x5F3759DF) - lax.shift_right_logical(i, 1)
    y = lax.bitcast_convert_type(i, jnp.float32)
    half_x = 0.5 * x
    for _ in range(2):
        y = y * (1.5 - half_x * y * y)
    return y


def _lane_sum(x, perms):
    for perm in perms:
        x = x + x.at[perm].get(mode="promise_in_bounds")
    return x


def _tree_add(vs):
    while len(vs) > 1:
        vs = [a + b for a, b in zip(vs[::2], vs[1::2])]
    return vs[0]


def _make_kernel(L, T, C, B):
    D = T * C
    CL = C // 128
    per_w = B // _NW
    n_chunks = per_w // _ROWS
    sub = C // _LANES

    mesh = plsc.VectorSubcoreMesh(core_axis_name="c", subcore_axis_name="s")

    @functools.partial(
        pl.kernel,
        mesh=mesh,
        out_type=jax.ShapeDtypeStruct((B, T, C), jnp.float32),
        scratch_types=[
            pltpu.VMEM((n_chunks, _ROWS), jnp.int32),
            pltpu.VMEM((_ROWS, D // 128, 128), jnp.float32),
            pltpu.VMEM((_ROWS, D // 128, 128), jnp.float32),
            pltpu.VMEM((_ROWS, T, C), jnp.float32),
            pltpu.VMEM((_ROWS, T, C), jnp.float32),
            pltpu.VMEM((C,), jnp.float32),
            pltpu.VMEM((C,), jnp.float32),
            pltpu.SemaphoreType.DMA,
            pltpu.SemaphoreType.DMA,
            pltpu.SemaphoreType.DMA,
            pltpu.SemaphoreType.DMA,
        ],
    )
    def fused_kernel(tab_hbm, idx_hbm, gamma_hbm, beta_hbm, out_hbm,
                     idx_v, in0, in1, st0, st1, gam_v, bet_v,
                     si0, si1, so0, so1):
        wid = lax.axis_index("s") * _NC + lax.axis_index("c")
        base = wid * per_w
        pltpu.sync_copy(idx_hbm.at[wid], idx_v)
        pltpu.sync_copy(gamma_hbm, gam_v)
        pltpu.sync_copy(beta_hbm, bet_v)
        lane = lax.iota(jnp.int32, _LANES)
        perms = tuple(lane ^ step for step in (8, 4, 2, 1))
        inv_n = jnp.float32(1.0 / C)

        inb = (in0, in1)
        stb = (st0, st1)
        sin = (si0, si1)
        sout = (so0, so1)

        def compute(src, dst):
            def per_rpair(rp, carry):
                r0 = 2 * rp
                r1 = r0 + 1

                def per_quad(th, carry2):
                    u0 = (2 * th) * CL
                    u1 = u0 + CL
                    zeros = jnp.zeros((_LANES,), jnp.float32)
                    init = (zeros,) * 8

                    def acc(j, carry3):
                        a00, q00, a01, q01, a10, q10, a11, q11 = carry3
                        u = j // 8
                        col = (j % 8) * _LANES
                        v00 = src[r0, u0 + u, pl.ds(col, _LANES)]
                        v01 = src[r0, u1 + u, pl.ds(col, _LANES)]
                        v10 = src[r1, u0 + u, pl.ds(col, _LANES)]
                        v11 = src[r1, u1 + u, pl.ds(col, _LANES)]
                        return (a00 + v00, q00 + v00 * v00,
                                a01 + v01, q01 + v01 * v01,
                                a10 + v10, q10 + v10 * v10,
                                a11 + v11, q11 + v11 * v11)

                    s00, q00, s01, q01, s10, q10, s11, q11 = lax.fori_loop(
                        0, sub, acc, init, unroll=4
                    )
                    m00 = _lane_sum(s00, perms) * inv_n
                    m01 = _lane_sum(s01, perms) * inv_n
                    m10 = _lane_sum(s10, perms) * inv_n
                    m11 = _lane_sum(s11, perms) * inv_n
                    r00 = _rsqrt16(_lane_sum(q00, perms) * inv_n - m00 * m00 + _EPS)
                    r01 = _rsqrt16(_lane_sum(q01, perms) * inv_n - m01 * m01 + _EPS)
                    r10 = _rsqrt16(_lane_sum(q10, perms) * inv_n - m10 * m10 + _EPS)
                    r11 = _rsqrt16(_lane_sum(q11, perms) * inv_n - m11 * m11 + _EPS)

                    def norm(j, carry3):
                        off = j * _LANES
                        u = j // 8
                        col = (j % 8) * _LANES
                        g = gam_v[pl.ds(off, _LANES)]
                        b = bet_v[pl.ds(off, _LANES)]
                        v00 = src[r0, u0 + u, pl.ds(col, _LANES)]
                        v01 = src[r0, u1 + u, pl.ds(col, _LANES)]
                        v10 = src[r1, u0 + u, pl.ds(col, _LANES)]
                        v11 = src[r1, u1 + u, pl.ds(col, _LANES)]
                        dst[r0, 2 * th, pl.ds(off, _LANES)] = (v00 - m00) * r00 * g + b
                        dst[r0, 2 * th + 1, pl.ds(off, _LANES)] = (v01 - m01) * r01 * g + b
                        dst[r1, 2 * th, pl.ds(off, _LANES)] = (v10 - m10) * r10 * g + b
                        dst[r1, 2 * th + 1, pl.ds(off, _LANES)] = (v11 - m11) * r11 * g + b
                        return carry3

                    lax.fori_loop(0, sub, norm, 0, unroll=4)
                    return carry2

                return lax.fori_loop(0, T // 2, per_quad, carry)

            lax.fori_loop(0, _ROWS // 2, per_rpair, 0)

        def wait_in(k):
            pltpu.make_async_copy(tab_hbm.at[idx_v.at[0]], inb[k], sin[k]).wait()

        def wait_out(k):
            pltpu.make_async_copy(
                stb[k], out_hbm.at[pl.ds(0, _ROWS)], sout[k]
            ).wait()

        def chunk_pair(cbase, first, last):
            for k in (0, 1):
                c = cbase + k
                wait_in(k)
                if not first:
                    wait_out(k)
                compute(inb[k], stb[k])
                pltpu.async_copy(
                    stb[k], out_hbm.at[pl.ds(base + c * _ROWS, _ROWS)], sout[k]
                )
                if not last:
                    pltpu.async_copy(
                        tab_hbm.at[idx_v.at[c + 2]], inb[k], sin[k]
                    )

        pltpu.async_copy(tab_hbm.at[idx_v.at[0]], in0, si0)
        pltpu.async_copy(tab_hbm.at[idx_v.at[1]], in1, si1)

        chunk_pair(0, first=True, last=False)

        def body(i, carry):
            chunk_pair(2 * i, first=False, last=False)
            return carry

        lax.fori_loop(1, n_chunks // 2 - 1, body, 0)

        chunk_pair(n_chunks - 2, first=False, last=True)
        wait_out(0)
        wait_out(1)

    return fused_kernel


def kernel(instructions, gamma, beta, idx_subject, idx_label):
    S, L, T, C = instructions.shape
    B = idx_label.shape[0]

    tab = jnp.reshape(instructions[idx_subject], (L, (T * C) // 128, 128))
    per_w = B // _NW
    idx = jnp.reshape(idx_label.astype(jnp.int32), (_NW, per_w // _ROWS, _ROWS))
    fn = _make_kernel(L, T, C, B)
    return fn(tab, idx, gamma, beta)
